# baseline (speedup 1.0000x reference)
import functools

import jax
import jax.numpy as jnp
from jax import lax
from jax.experimental import pallas as pl
from jax.experimental.pallas import tpu as pltpu
from jax.experimental.pallas import tpu_sc as plsc

_N = 10000
_D = 128
_H1 = 32
_H2 = 16
_E = 320000

_NC, _NS = 2, 16
_NW = _NC * _NS
_EPW = 10240
_EPAD = _NW * _EPW
_CHUNK = 1024
_NB = _CHUNK // 128
_NCHUNKS = _EPW // _CHUNK
_RPW = _EPW // 128
_NP = 10240
_RPS = _NP // _NS


def _spmm_body(support, srcs, dsts, ws, zeros, out,
               src_v0, src_v1, dst_v0, dst_v1, w_v0, w_v1,
               rows_v0, rows_v1, acc, sup_sh,
               gsem0, gsem1, ssem0, ssem1):
    c = lax.axis_index("c")
    s = lax.axis_index("s")
    wid = s * _NC + c

    src_v = [src_v0, src_v1]
    dst_v = [dst_v0, dst_v1]
    w_v = [w_v0, w_v1]
    rows_v = [rows_v0, rows_v1]
    gsem = [gsem0, gsem1]
    ssem = [ssem0, ssem1]

    @pl.when(s == 0)
    def _():
        pltpu.sync_copy(zeros, acc)

    @pl.when(s == 1)
    def _():
        pltpu.sync_copy(support, sup_sh)

    plsc.subcore_barrier()

    def load_idx(ci, bi):
        rb = wid * _RPW + ci * _NB
        pltpu.sync_copy(srcs.at[pl.ds(rb, _NB)], src_v[bi])
        pltpu.sync_copy(dsts.at[pl.ds(rb, _NB)], dst_v[bi])
        pltpu.sync_copy(ws.at[pl.ds(rb, _NB)], w_v[bi])

    def fire_gathers(bi):
        return [pltpu.async_copy(sup_sh.at[src_v[bi].at[b]],
                                 rows_v[bi].at[b], gsem[bi])
                for b in range(_NB)]

    def fire_scatters(bi):
        return [pltpu.async_copy(rows_v[bi].at[b],
                                 acc.at[dst_v[bi].at[b]], ssem[bi], add=True)
                for b in range(_NB)]

    def scale(bi):
        rv = rows_v[bi]
        wv = w_v[bi]

        def scale_body(t, carry2):
            b = t // 8
            g = (t % 8) * 16
            w16 = wv[b, pl.ds(g, 16)]
            for j in range(16):
                wj = w16[j]
                r0 = rv[b, g + j, pl.ds(0, 16)]
                r1 = rv[b, g + j, pl.ds(16, 16)]
                rv[b, g + j, pl.ds(0, 16)] = r0 * wj
                rv[b, g + j, pl.ds(16, 16)] = r1 * wj
            return carry2

        lax.fori_loop(0, _NB * 8, scale_body, 0)

    load_idx(0, 0)
    gcps = [fire_gathers(0), None]
    scps = [None, None]
    for ci in range(_NCHUNKS):
        bi = ci & 1
        if ci + 1 < _NCHUNKS:
            if scps[1 - bi] is not None:
                for cp in scps[1 - bi]:
                    cp.wait()
            load_idx(ci + 1, 1 - bi)
            gcps[1 - bi] = fire_gathers(1 - bi)
        for cp in gcps[bi]:
            cp.wait()
        scale(bi)
        scps[bi] = fire_scatters(bi)

    for sc in scps:
        if sc is not None:
            for cp in sc:
                cp.wait()

    plsc.subcore_barrier()
    pltpu.sync_copy(acc.at[pl.ds(s * _RPS, _RPS)],
                    out.at[c, pl.ds(s * _RPS, _RPS)])


_spmm = functools.partial(
    pl.kernel,
    out_type=jax.ShapeDtypeStruct((_NC, _NP, _H1), jnp.float32),
    mesh=plsc.VectorSubcoreMesh(core_axis_name="c", subcore_axis_name="s"),
    scratch_types=[
        pltpu.VMEM((_NB, 128), jnp.int32),
        pltpu.VMEM((_NB, 128), jnp.int32),
        pltpu.VMEM((_NB, 128), jnp.int32),
        pltpu.VMEM((_NB, 128), jnp.int32),
        pltpu.VMEM((_NB, 128), jnp.float32),
        pltpu.VMEM((_NB, 128), jnp.float32),
        pltpu.VMEM((_NB, 128, _H1), jnp.float32),
        pltpu.VMEM((_NB, 128, _H1), jnp.float32),
        pltpu.VMEM_SHARED((_NP, _H1), jnp.float32),
        pltpu.VMEM_SHARED((_N, _H1), jnp.float32),
        pltpu.SemaphoreType.DMA,
        pltpu.SemaphoreType.DMA,
        pltpu.SemaphoreType.DMA,
        pltpu.SemaphoreType.DMA,
    ],
    compiler_params=pltpu.CompilerParams(use_tc_tiling_on_sc=False),
)(_spmm_body)


def _mm1_body(x_ref, w_ref, o_ref):
    o_ref[...] = jnp.dot(x_ref[...], w_ref[...],
                         preferred_element_type=jnp.float32)


_mm1 = pl.pallas_call(
    _mm1_body,
    grid=(10,),
    in_specs=[pl.BlockSpec((1000, _D), lambda i: (i, 0)),
              pl.BlockSpec((_D, _H1), lambda i: (0, 0))],
    out_specs=pl.BlockSpec((1000, _H1), lambda i: (i, 0)),
    out_shape=jax.ShapeDtypeStruct((_N, _H1), jnp.float32),
)


def _mm2_body(p_ref, w_ref, o_ref):
    h = jnp.maximum(p_ref[0] + p_ref[1], 0.0)
    o_ref[...] = jnp.dot(h, w_ref[...], preferred_element_type=jnp.float32)


_mm2 = pl.pallas_call(
    _mm2_body,
    grid=(10,),
    in_specs=[pl.BlockSpec((2, 1000, _H1), lambda i: (0, i, 0)),
              pl.BlockSpec((_H1, 2 * _H2), lambda i: (0, 0))],
    out_specs=pl.BlockSpec((1000, 2 * _H2), lambda i: (i, 0)),
    out_shape=jax.ShapeDtypeStruct((_N, 2 * _H2), jnp.float32),
)


def _comb_body(q_ref, mu_ref, lv_ref):
    z = q_ref[0] + q_ref[1]
    mu_ref[...] = z[:, :_H2]
    lv_ref[...] = z[:, _H2:]


_comb = pl.pallas_call(
    _comb_body,
    grid=(10,),
    in_specs=[pl.BlockSpec((2, 1000, 2 * _H2), lambda i: (0, i, 0))],
    out_specs=[pl.BlockSpec((1000, _H2), lambda i: (i, 0)),
               pl.BlockSpec((1000, _H2), lambda i: (i, 0))],
    out_shape=[jax.ShapeDtypeStruct((_N, _H2), jnp.float32),
               jax.ShapeDtypeStruct((_N, _H2), jnp.float32)],
)

_BM = 512
_BN = 2048


def _dec_body(qa_ref, qb_ref, o_ref):
    a = (qa_ref[0, :, :_H2] + qa_ref[1, :, :_H2]).astype(jnp.bfloat16)
    b = (qb_ref[0, :, :_H2] + qb_ref[1, :, :_H2]).astype(jnp.bfloat16)
    o_ref[...] = lax.dot_general(a, b, (((1,), (1,)), ((), ())),
                                 preferred_element_type=jnp.float32)


_dec = pl.pallas_call(
    _dec_body,
    grid=(pl.cdiv(_N, _BM), pl.cdiv(_N, _BN)),
    in_specs=[pl.BlockSpec((2, _BM, 2 * _H2), lambda i, j: (0, i, 0)),
              pl.BlockSpec((2, _BN, 2 * _H2), lambda i, j: (0, j, 0))],
    out_specs=pl.BlockSpec((_BM, _BN), lambda i, j: (i, j)),
    out_shape=jax.ShapeDtypeStruct((_N, _N), jnp.float32),
)


def kernel(x, edge_index, edge_weight, W1, W2, W3):
    dst = edge_index[0]
    src = edge_index[1]
    pad = _EPAD - _E
    srcs = jnp.pad(src, (0, pad)).reshape(_EPAD // 128, 128)
    dsts = jnp.pad(dst, (0, pad)).reshape(_EPAD // 128, 128)
    ws = jnp.pad(edge_weight, (0, pad)).reshape(_EPAD // 128, 128)
    zeros = jnp.zeros((_NP, _H1), jnp.float32)

    support1 = _mm1(x, W1)
    p = _spmm(support1, srcs, dsts, ws, zeros)
    h23 = _mm2(p, jnp.concatenate([W2, W3], axis=1))
    q = _spmm(h23, srcs, dsts, ws, zeros)
    mu, logvar = _comb(q)
    adj_recon = _dec(q, q)
    return (adj_recon, mu, mu, logvar)


# speedup vs baseline: 1.1101x; 1.1101x over previous
import functools

import jax
import jax.numpy as jnp
from jax import lax
from jax.experimental import pallas as pl
from jax.experimental.pallas import tpu as pltpu
from jax.experimental.pallas import tpu_sc as plsc

_N = 10000
_D = 128
_H1 = 32
_H2 = 16
_E = 320000

_NC, _NS = 2, 16
_NW = _NC * _NS
_EPW = 10240
_EPAD = _NW * _EPW
_CHUNK = 1024
_NB = _CHUNK // 128
_NCHUNKS = _EPW // _CHUNK
_RPW = _EPW // 128
_NP = 10240
_RPS = _NP // _NS


def _spmm_body(support, srcs, dsts, ws, zeros, out,
               src_v0, src_v1, dst_v0, dst_v1, w_v0, w_v1,
               rows_v0, rows_v1, acc, sup_sh,
               gsem0, gsem1, ssem0, ssem1):
    c = lax.axis_index("c")
    s = lax.axis_index("s")
    wid = s * _NC + c

    src_v = [src_v0, src_v1]
    dst_v = [dst_v0, dst_v1]
    w_v = [w_v0, w_v1]
    rows_v = [rows_v0, rows_v1]
    gsem = [gsem0, gsem1]
    ssem = [ssem0, ssem1]

    @pl.when(s == 0)
    def _():
        pltpu.sync_copy(zeros, acc)

    @pl.when(s == 1)
    def _():
        pltpu.sync_copy(support, sup_sh)

    plsc.subcore_barrier()

    def load_idx(ci, bi):
        rb = wid * _RPW + ci * _NB
        pltpu.sync_copy(srcs.at[pl.ds(rb, _NB)], src_v[bi])
        pltpu.sync_copy(dsts.at[pl.ds(rb, _NB)], dst_v[bi])
        pltpu.sync_copy(ws.at[pl.ds(rb, _NB)], w_v[bi])

    def fire_gathers(bi):
        return [pltpu.async_copy(sup_sh.at[src_v[bi].at[b]],
                                 rows_v[bi].at[b], gsem[bi])
                for b in range(_NB)]

    def fire_scatters(bi):
        return [pltpu.async_copy(rows_v[bi].at[b],
                                 acc.at[dst_v[bi].at[b]], ssem[bi], add=True)
                for b in range(_NB)]

    def scale(bi):
        rv = rows_v[bi]
        wv = w_v[bi]

        def scale_body(t, carry2):
            b = t // 8
            g = (t % 8) * 16
            w16 = wv[b, pl.ds(g, 16)]
            for j in range(16):
                wj = w16[j]
                r0 = rv[b, g + j, pl.ds(0, 16)]
                r1 = rv[b, g + j, pl.ds(16, 16)]
                rv[b, g + j, pl.ds(0, 16)] = r0 * wj
                rv[b, g + j, pl.ds(16, 16)] = r1 * wj
            return carry2

        lax.fori_loop(0, _NB * 8, scale_body, 0)

    load_idx(0, 0)
    gcps = [fire_gathers(0), None]
    scps = [None, None]
    for ci in range(_NCHUNKS):
        bi = ci & 1
        if ci + 1 < _NCHUNKS:
            if scps[1 - bi] is not None:
                for cp in scps[1 - bi]:
                    cp.wait()
            load_idx(ci + 1, 1 - bi)
            gcps[1 - bi] = fire_gathers(1 - bi)
        for cp in gcps[bi]:
            cp.wait()
        scale(bi)
        scps[bi] = fire_scatters(bi)

    for sc in scps:
        if sc is not None:
            for cp in sc:
                cp.wait()

    plsc.subcore_barrier()
    pltpu.sync_copy(acc.at[pl.ds(s * _RPS, _RPS)],
                    out.at[c, pl.ds(s * _RPS, _RPS)])


_spmm = functools.partial(
    pl.kernel,
    out_type=jax.ShapeDtypeStruct((_NC, _NP, _H1), jnp.float32),
    mesh=plsc.VectorSubcoreMesh(core_axis_name="c", subcore_axis_name="s"),
    scratch_types=[
        pltpu.VMEM((_NB, 128), jnp.int32),
        pltpu.VMEM((_NB, 128), jnp.int32),
        pltpu.VMEM((_NB, 128), jnp.int32),
        pltpu.VMEM((_NB, 128), jnp.int32),
        pltpu.VMEM((_NB, 128), jnp.float32),
        pltpu.VMEM((_NB, 128), jnp.float32),
        pltpu.VMEM((_NB, 128, _H1), jnp.float32),
        pltpu.VMEM((_NB, 128, _H1), jnp.float32),
        pltpu.VMEM_SHARED((_NP, _H1), jnp.float32),
        pltpu.VMEM_SHARED((_N, _H1), jnp.float32),
        pltpu.SemaphoreType.DMA,
        pltpu.SemaphoreType.DMA,
        pltpu.SemaphoreType.DMA,
        pltpu.SemaphoreType.DMA,
    ],
    compiler_params=pltpu.CompilerParams(use_tc_tiling_on_sc=False),
)(_spmm_body)


def _mm1_body(x_ref, w_ref, o_ref):
    o_ref[...] = jnp.dot(x_ref[...], w_ref[...],
                         preferred_element_type=jnp.float32)


_mm1 = pl.pallas_call(
    _mm1_body,
    grid=(10,),
    in_specs=[pl.BlockSpec((1000, _D), lambda i: (i, 0)),
              pl.BlockSpec((_D, _H1), lambda i: (0, 0))],
    out_specs=pl.BlockSpec((1000, _H1), lambda i: (i, 0)),
    out_shape=jax.ShapeDtypeStruct((_N, _H1), jnp.float32),
)


def _mm2_body(p_ref, w_ref, o_ref):
    h = jnp.maximum(p_ref[0] + p_ref[1], 0.0)
    o_ref[...] = jnp.dot(h, w_ref[...], preferred_element_type=jnp.float32)


_mm2 = pl.pallas_call(
    _mm2_body,
    grid=(10,),
    in_specs=[pl.BlockSpec((2, 1000, _H1), lambda i: (0, i, 0)),
              pl.BlockSpec((_H1, 2 * _H2), lambda i: (0, 0))],
    out_specs=pl.BlockSpec((1000, 2 * _H2), lambda i: (i, 0)),
    out_shape=jax.ShapeDtypeStruct((_N, 2 * _H2), jnp.float32),
)


def _comb_body(q_ref, mu_ref, lv_ref):
    z = q_ref[0] + q_ref[1]
    mu_ref[...] = z[:, :_H2]
    lv_ref[...] = z[:, _H2:]


_comb = pl.pallas_call(
    _comb_body,
    grid=(10,),
    in_specs=[pl.BlockSpec((2, 1000, 2 * _H2), lambda i: (0, i, 0))],
    out_specs=[pl.BlockSpec((1000, _H2), lambda i: (i, 0)),
               pl.BlockSpec((1000, _H2), lambda i: (i, 0))],
    out_shape=[jax.ShapeDtypeStruct((_N, _H2), jnp.float32),
               jax.ShapeDtypeStruct((_N, _H2), jnp.float32)],
)

_BM = 1024
_BN = 1024


def _dec_body(a_ref, b_ref, o_ref):
    o_ref[...] = lax.dot_general(a_ref[...], b_ref[...],
                                 (((1,), (1,)), ((), ())),
                                 preferred_element_type=jnp.float32)


_dec = pl.pallas_call(
    _dec_body,
    grid=(pl.cdiv(_N, _BM), pl.cdiv(_N, _BN)),
    in_specs=[pl.BlockSpec((_BM, _H2), lambda i, j: (i, 0)),
              pl.BlockSpec((_BN, _H2), lambda i, j: (j, 0))],
    out_specs=pl.BlockSpec((_BM, _BN), lambda i, j: (i, j)),
    out_shape=jax.ShapeDtypeStruct((_N, _N), jnp.float32),
)


def kernel(x, edge_index, edge_weight, W1, W2, W3):
    dst = edge_index[0]
    src = edge_index[1]
    pad = _EPAD - _E
    srcs = jnp.pad(src, (0, pad)).reshape(_EPAD // 128, 128)
    dsts = jnp.pad(dst, (0, pad)).reshape(_EPAD // 128, 128)
    ws = jnp.pad(edge_weight, (0, pad)).reshape(_EPAD // 128, 128)
    zeros = jnp.zeros((_NP, _H1), jnp.float32)

    support1 = _mm1(x, W1)
    p = _spmm(support1, srcs, dsts, ws, zeros)
    h23 = _mm2(p, jnp.concatenate([W2, W3], axis=1))
    q = _spmm(h23, srcs, dsts, ws, zeros)
    mu, logvar = _comb(q)
    adj_recon = _dec(mu, mu)
    return (adj_recon, mu, mu, logvar)
